# Initial kernel scaffold; baseline (speedup 1.0000x reference)
import jax
import jax.numpy as jnp
from jax.experimental import pallas as pl


def kernel(x, token_emb, pos_emb):
    raise NotImplementedError("write your pallas kernel here")


# speedup vs baseline: 3.1171x; 3.1171x over previous
import functools

import jax
import jax.numpy as jnp
from jax import lax
from jax.experimental import pallas as pl
from jax.experimental.pallas import tpu as pltpu, tpu_sc as plsc

VOCAB_SIZE = 100000
MAX_LEN = 200
EMBED_DIM = 64
BATCH = 4096

_NC = 2
_NS = 16
_NW = _NC * _NS
_SEQ_PER_W = BATCH // _NW
_HALF = MAX_LEN // 2


def _body(x_hbm, tok_hbm, pos_hbm, out_hbm, pos_v, idx_v, rows_v, sem):
    wid = lax.axis_index("s") * _NC + lax.axis_index("c")
    base = wid * _SEQ_PER_W

    pltpu.sync_copy(pos_hbm, pos_v)

    def seq_step(i, carry):
        seq = base + i
        pltpu.sync_copy(x_hbm.at[seq], idx_v)

        def row_copy(r, c):
            for j in range(EMBED_DIM // 16):
                rows_v[r, pl.ds(j * 16, 16)] = pos_v[r, pl.ds(j * 16, 16)]
            return c

        lax.fori_loop(0, MAX_LEN, row_copy, 0)
        c0 = pltpu.async_copy(tok_hbm.at[idx_v.at[0]],
                              rows_v.at[pl.ds(0, _HALF)], sem, add=True)
        c1 = pltpu.async_copy(tok_hbm.at[idx_v.at[1]],
                              rows_v.at[pl.ds(_HALF, _HALF)], sem, add=True)
        c0.wait()
        c1.wait()
        pltpu.sync_copy(rows_v, out_hbm.at[seq])
        return carry

    lax.fori_loop(0, _SEQ_PER_W, seq_step, 0)


@jax.jit
def kernel(x, token_emb, pos_emb):
    x3 = x.astype(jnp.int32).reshape(BATCH, 2, _HALF)
    mesh = plsc.VectorSubcoreMesh(core_axis_name="c", subcore_axis_name="s")
    k = functools.partial(
        pl.kernel,
        out_type=jax.ShapeDtypeStruct((BATCH, MAX_LEN, EMBED_DIM), jnp.float32),
        mesh=mesh,
        scratch_types=[
            pltpu.VMEM((MAX_LEN, EMBED_DIM), jnp.float32),
            pltpu.VMEM((2, _HALF), jnp.int32),
            pltpu.VMEM((MAX_LEN, EMBED_DIM), jnp.float32),
            pltpu.SemaphoreType.DMA,
        ],
        compiler_params=pltpu.CompilerParams(use_tc_tiling_on_sc=False),
    )(_body)
    return k(x3, token_emb, pos_emb)


# speedup vs baseline: 3.5760x; 1.1472x over previous
import functools

import jax
import jax.numpy as jnp
from jax import lax
from jax.experimental import pallas as pl
from jax.experimental.pallas import tpu as pltpu, tpu_sc as plsc

VOCAB_SIZE = 100000
MAX_LEN = 200
EMBED_DIM = 64
BATCH = 4096

_NC = 2
_NS = 16
_NW = _NC * _NS
_SEQ_PER_W = BATCH // _NW
_K = 2
_STEPS = _SEQ_PER_W // _K
_HALF = 100
_GPS = _K * MAX_LEN // _HALF
_ROWS = _K * MAX_LEN


def _body(x_hbm, tok_hbm, pos_hbm, out_hbm,
          pos_v, idx_v, rows_a, rows_b, sg_a, sg_b, sw_a, sw_b):
    wid = lax.axis_index("s") * _NC + lax.axis_index("c")
    sbase = wid * _STEPS

    pltpu.sync_copy(pos_hbm, pos_v)
    pltpu.sync_copy(x_hbm.at[pl.ds(sbase, _STEPS)], idx_v)

    def copy_pos(rows):
        def cp(r, c):
            for rr in range(2):
                for j in range(EMBED_DIM // 16):
                    v = pos_v[2 * r + rr, pl.ds(j * 16, 16)]
                    for h in range(_K):
                        rows[h * MAX_LEN + 2 * r + rr, pl.ds(j * 16, 16)] = v
            return c

        lax.fori_loop(0, MAX_LEN // 2, cp, 0, unroll=2)

    def fire_gathers(t, rows, sg):
        for j in range(_GPS):
            pltpu.async_copy(tok_hbm.at[idx_v.at[t, j]],
                             rows.at[pl.ds(j * _HALF, _HALF)], sg, add=True)

    def drain_gathers(t, rows, sg):
        for j in range(_GPS):
            pltpu.make_async_copy(tok_hbm.at[idx_v.at[t, j]],
                                  rows.at[pl.ds(j * _HALF, _HALF)], sg).wait()

    def fire_writeback(t, rows, sw):
        pltpu.async_copy(rows, out_hbm.at[sbase + t], sw)

    def drain_writeback(t, rows, sw):
        pltpu.make_async_copy(rows, out_hbm.at[sbase + t], sw).wait()

    bufs = ((rows_a, sg_a, sw_a), (rows_b, sg_b, sw_b))

    def step(u, carry):
        for b, (rows, sg, sw) in enumerate(bufs):
            t = 2 * u + b
            other = bufs[1 - b]

            @pl.when(u > 0)
            def _():
                drain_writeback(t - 2, rows, sw)

            copy_pos(rows)
            fire_gathers(t, rows, sg)

            def drain_other():
                o_rows, o_sg, o_sw = other
                drain_gathers(t - 1, o_rows, o_sg)
                fire_writeback(t - 1, o_rows, o_sw)

            if b == 0:
                @pl.when(u > 0)
                def _():
                    drain_other()
            else:
                drain_other()
        return carry

    lax.fori_loop(0, _STEPS // 2, step, 0)

    t_last = _STEPS - 1
    drain_gathers(t_last, rows_b, sg_b)
    fire_writeback(t_last, rows_b, sw_b)
    drain_writeback(t_last, rows_b, sw_b)
    drain_writeback(t_last - 1, rows_a, sw_a)


@jax.jit
def kernel(x, token_emb, pos_emb):
    x4 = x.astype(jnp.int32).reshape(BATCH // _K, _GPS, _HALF)
    mesh = plsc.VectorSubcoreMesh(core_axis_name="c", subcore_axis_name="s")
    k = functools.partial(
        pl.kernel,
        out_type=jax.ShapeDtypeStruct((BATCH // _K, _ROWS, EMBED_DIM),
                                      jnp.float32),
        mesh=mesh,
        scratch_types=[
            pltpu.VMEM((MAX_LEN, EMBED_DIM), jnp.float32),
            pltpu.VMEM((_STEPS, _GPS, _HALF), jnp.int32),
            pltpu.VMEM((_ROWS, EMBED_DIM), jnp.float32),
            pltpu.VMEM((_ROWS, EMBED_DIM), jnp.float32),
            pltpu.SemaphoreType.DMA,
            pltpu.SemaphoreType.DMA,
            pltpu.SemaphoreType.DMA,
            pltpu.SemaphoreType.DMA,
        ],
        compiler_params=pltpu.CompilerParams(use_tc_tiling_on_sc=False),
    )(_body)
    out = k(x4, token_emb, pos_emb)
    return out.reshape(BATCH, MAX_LEN, EMBED_DIM)
